# baseline (speedup 1.0000x reference)
import jax
import jax.numpy as jnp
from jax.experimental import pallas as pl

B = 16384
D = 128
N_NT = 16
R = 16
G = B // D
GQ = 64


def _body(x_ref, nt_ref, w_ref, idx_ref, o_ref):
    f32 = jnp.float32
    i = pl.program_id(0)
    idx = idx_ref[...]
    w = w_ref[...]
    nt_ids = jax.lax.broadcasted_iota(jnp.int32, (R, N_NT), 1)
    sel = (idx.reshape(R, 1) == nt_ids).astype(f32)
    s = (w.reshape(R, 1) * sel).sum(axis=0, keepdims=True)
    blk = nt_ref[:, pl.ds(i * GQ, GQ), :]
    contrib = jnp.zeros((GQ, D), f32)
    for n in range(N_NT):
        contrib = contrib + blk[n] * jnp.broadcast_to(s[0:1, n:n + 1], (GQ, D))
    g2 = 0.1 + 1.9 * jax.nn.sigmoid(contrib)
    o_ref[...] = x_ref[...] * g2[:, :, None]


@jax.jit
def kernel(x, nt_levels, w, idx):
    x3 = x.reshape(G, D, D)
    ntt3 = nt_levels.T.reshape(N_NT, G, D)
    out = pl.pallas_call(
        _body,
        grid=(G // GQ,),
        in_specs=[
            pl.BlockSpec((GQ, D, D), lambda i: (i, 0, 0)),
            pl.BlockSpec((N_NT, G, D), lambda i: (0, 0, 0)),
            pl.BlockSpec((1, R), lambda i: (0, 0)),
            pl.BlockSpec((1, R), lambda i: (0, 0)),
        ],
        out_specs=pl.BlockSpec((GQ, D, D), lambda i: (i, 0, 0)),
        out_shape=jax.ShapeDtypeStruct((G, D, D), jnp.float32),
    )(x3, ntt3, w.reshape(1, R), idx.reshape(1, R))
    return out.reshape(B, D)
